# baseline (speedup 1.0000x reference)
import functools

import jax
import jax.numpy as jnp
from jax import lax
from jax.experimental import pallas as pl
from jax.experimental.pallas import tpu as pltpu
from jax.experimental.pallas import tpu_sc as plsc

NW = 32
C = 128
PAIR = 1
NBUF = 7
G = 6


@functools.lru_cache(maxsize=None)
def _build(h, b, d):
    nch = h
    npair = nch // PAIR
    mesh = plsc.VectorSubcoreMesh(core_axis_name="c", subcore_axis_name="s")

    @functools.partial(
        pl.kernel,
        out_type=jax.ShapeDtypeStruct((h, b, d), jnp.float32),
        mesh=mesh,
        scratch_types=[
            pltpu.VMEM((nch, C), jnp.int32),
            pltpu.VMEM((NBUF, PAIR, C, d), jnp.float32),
            pltpu.SemaphoreType.DMA((NBUF,)),
            pltpu.SemaphoreType.DMA((NBUF,)),
        ],
    )
    def gather_kernel(idx_hbm, table_hbm, out_hbm, idx_v, rows_v, gsem, osem):
        wid = lax.axis_index("s") * 2 + lax.axis_index("c")
        col0 = wid * C
        pltpu.sync_copy(idx_hbm.at[:, pl.ds(col0, C)], idx_v)

        def gather_start(p, buf):
            for j in range(PAIR):
                pltpu.async_copy(table_hbm.at[idx_v.at[p * PAIR + j]],
                                 rows_v.at[buf, j], gsem.at[buf])

        def gather_wait(p, buf):
            for j in range(PAIR):
                pltpu.make_async_copy(table_hbm.at[idx_v.at[p * PAIR + j]],
                                      rows_v.at[buf, j], gsem.at[buf]).wait()

        def out_ref(p, buf):
            return rows_v.at[buf], out_hbm.at[pl.ds(p * PAIR, PAIR),
                                              pl.ds(col0, C)]

        def store_start(p, buf):
            src, dst = out_ref(p, buf)
            pltpu.async_copy(src, dst, osem.at[buf])

        def store_wait(p, buf):
            src, dst = out_ref(p, buf)
            pltpu.make_async_copy(src, dst, osem.at[buf]).wait()

        def slot(p, buf, wait_store, new_gather):
            gather_wait(p, buf)
            store_start(p, buf)
            if new_gather:
                lbuf = (buf + G) % NBUF
                if wait_store:
                    store_wait(p + G - NBUF, lbuf)
                gather_start(p + G, lbuf)

        for p in range(G):
            gather_start(p, p)

        for p in range(NBUF - G):
            slot(p, p, False, True)

        n_steady = ((npair - G) - (NBUF - G)) // NBUF * NBUF

        @pl.loop(NBUF - G, NBUF - G + n_steady, step=NBUF)
        def _(p0):
            for i in range(NBUF):
                slot(p0 + i, (NBUF - G + i) % NBUF, True, True)

        for p in range(NBUF - G + n_steady, npair - G):
            slot(p, p % NBUF, True, True)

        for p in range(npair - G, npair):
            slot(p, p % NBUF, False, False)
        for p in range(npair - NBUF, npair):
            store_wait(p, p % NBUF)

    return gather_kernel


def kernel(x, table):
    b, h = x.shape
    d = table.shape[1]
    assert h % PAIR == 0 and b == NW * C
    out = _build(h, b, d)(x.T, table)
    return out.transpose(1, 0, 2)


# speedup vs baseline: 1.0035x; 1.0035x over previous
import functools

import jax
import jax.numpy as jnp
from jax import lax
from jax.experimental import pallas as pl
from jax.experimental.pallas import tpu as pltpu
from jax.experimental.pallas import tpu_sc as plsc

NW = 32
C = 128
PAIR = 1
NBUF = 7
G = 5
IDX_HEAD = 8


@functools.lru_cache(maxsize=None)
def _build(h, b, d):
    nch = h
    npair = nch // PAIR
    mesh = plsc.VectorSubcoreMesh(core_axis_name="c", subcore_axis_name="s")

    @functools.partial(
        pl.kernel,
        out_type=jax.ShapeDtypeStruct((h, b, d), jnp.float32),
        mesh=mesh,
        scratch_types=[
            pltpu.VMEM((nch, C), jnp.int32),
            pltpu.VMEM((NBUF, PAIR, C, d), jnp.float32),
            pltpu.SemaphoreType.DMA((NBUF,)),
            pltpu.SemaphoreType.DMA((NBUF,)),
            pltpu.SemaphoreType.DMA,
        ],
    )
    def gather_kernel(idx_hbm, table_hbm, out_hbm, idx_v, rows_v, gsem, osem,
                      isem):
        wid = lax.axis_index("s") * 2 + lax.axis_index("c")
        col0 = wid * C
        pltpu.sync_copy(idx_hbm.at[pl.ds(0, IDX_HEAD), pl.ds(col0, C)],
                        idx_v.at[pl.ds(0, IDX_HEAD)])
        idx_tail = pltpu.make_async_copy(
            idx_hbm.at[pl.ds(IDX_HEAD, nch - IDX_HEAD), pl.ds(col0, C)],
            idx_v.at[pl.ds(IDX_HEAD, nch - IDX_HEAD)], isem)
        idx_tail.start()

        def gather_start(p, buf):
            for j in range(PAIR):
                pltpu.async_copy(table_hbm.at[idx_v.at[p * PAIR + j]],
                                 rows_v.at[buf, j], gsem.at[buf])

        def gather_wait(p, buf):
            for j in range(PAIR):
                pltpu.make_async_copy(table_hbm.at[idx_v.at[p * PAIR + j]],
                                      rows_v.at[buf, j], gsem.at[buf]).wait()

        def out_ref(p, buf):
            return rows_v.at[buf], out_hbm.at[pl.ds(p * PAIR, PAIR),
                                              pl.ds(col0, C)]

        def store_start(p, buf):
            src, dst = out_ref(p, buf)
            pltpu.async_copy(src, dst, osem.at[buf])

        def store_wait(p, buf):
            src, dst = out_ref(p, buf)
            pltpu.make_async_copy(src, dst, osem.at[buf]).wait()

        def slot(p, buf, wait_store, new_gather):
            gather_wait(p, buf)
            store_start(p, buf)
            if new_gather:
                lbuf = (buf + G) % NBUF
                if wait_store:
                    store_wait(p + G - NBUF, lbuf)
                gather_start(p + G, lbuf)

        for p in range(G):
            gather_start(p, p)

        for p in range(NBUF - G):
            slot(p, p, False, True)

        idx_tail.wait()

        n_steady = ((npair - G) - (NBUF - G)) // NBUF * NBUF

        @pl.loop(NBUF - G, NBUF - G + n_steady, step=NBUF)
        def _(p0):
            for i in range(NBUF):
                slot(p0 + i, (NBUF - G + i) % NBUF, True, True)

        for p in range(NBUF - G + n_steady, npair - G):
            slot(p, p % NBUF, True, True)

        for p in range(npair - G, npair):
            slot(p, p % NBUF, False, False)
        for p in range(npair - NBUF, npair):
            store_wait(p, p % NBUF)

    return gather_kernel


def kernel(x, table):
    b, h = x.shape
    d = table.shape[1]
    assert h % PAIR == 0 and b == NW * C
    out = _build(h, b, d)(x.T, table)
    return out.transpose(1, 0, 2)
